# baseline (device time: 162353 ns/iter reference)
import functools

import jax
import jax.numpy as jnp
from jax import lax
from jax.experimental import pallas as pl
from jax.experimental.pallas import tpu as pltpu

N_DEV = 8
SQ = 2048
SKV = 2048
D_MODEL = 1024
H_PER = 8
DH = 128
SCALE = 0.08838834764831843
CHUNK = SQ // N_DEV


QT = 256
N_QT = SQ // QT
KT = 256
N_KT = SKV // KT


def _key_tiles(qt: int) -> list[int]:
    if qt == 0:
        return list(range(N_KT))
    tiles = {0, qt - 1, qt, qt + 1}
    return sorted(t for t in tiles if 0 <= t < N_KT)


def _attn_body(col_ref, x_ref, wq_ref, k_hbm, v_hbm, ctx_ref, kv_buf, kv_sems):
    del col_ref
    h = pl.program_id(0)
    slot = lax.rem(h, 2)

    def kv_copy(head, sl):
        return (
            pltpu.make_async_copy(
                k_hbm.at[0, :, head, :], kv_buf.at[sl, 0], kv_sems.at[sl, 0]
            ),
            pltpu.make_async_copy(
                v_hbm.at[0, :, head, :], kv_buf.at[sl, 1], kv_sems.at[sl, 1]
            ),
        )

    @pl.when(h == 0)
    def _():
        for c in kv_copy(h, slot):
            c.start()

    @pl.when(h + 1 < H_PER)
    def _():
        for c in kv_copy(h + 1, lax.rem(h + 1, 2)):
            c.start()

    xm = x_ref[0]
    q = jnp.dot(xm, wq_ref[...], preferred_element_type=jnp.float32)
    for c in kv_copy(h, slot):
        c.wait()
    k = kv_buf[slot, 0]
    v = kv_buf[slot, 1]
    for qt in range(N_QT):
        tiles = _key_tiles(qt)
        q_t = q[qt * QT:(qt + 1) * QT, :]
        k_sel = jnp.concatenate([k[t * KT:(t + 1) * KT, :] for t in tiles], 0)
        v_sel = jnp.concatenate([v[t * KT:(t + 1) * KT, :] for t in tiles], 0)
        s = lax.dot_general(
            q_t, k_sel, (((1,), (1,)), ((), ())),
            preferred_element_type=jnp.float32,
        ) * SCALE
        qi = qt * QT + lax.broadcasted_iota(jnp.int32, s.shape, 0)
        ki = jnp.concatenate(
            [t * KT + lax.broadcasted_iota(jnp.int32, (QT, KT), 1) for t in tiles],
            axis=1,
        )
        mask = (jnp.abs(qi - ki) <= 128) | (ki < 32) | (qi < 32)
        s = jnp.where(mask, s, -1e9)
        m = jnp.max(s, axis=1, keepdims=True)
        w = jnp.exp(s - m)
        w = w / jnp.sum(w, axis=1, keepdims=True)
        ctx_ref[qt * QT:(qt + 1) * QT, :] = jnp.dot(
            w, v_sel, preferred_element_type=jnp.float32
        ).astype(jnp.bfloat16)


def _attention(x, Wq, K_ext, V_ext, col_base):
    return pl.pallas_call(
        _attn_body,
        grid_spec=pltpu.PrefetchScalarGridSpec(
            num_scalar_prefetch=1,
            grid=(H_PER,),
            in_specs=[
                pl.BlockSpec((1, SQ, D_MODEL), lambda h, c: (0, 0, 0)),
                pl.BlockSpec((D_MODEL, DH), lambda h, c: (0, c[0] + h)),
                pl.BlockSpec(memory_space=pltpu.MemorySpace.HBM),
                pl.BlockSpec(memory_space=pltpu.MemorySpace.HBM),
            ],
            out_specs=pl.BlockSpec((SQ, DH), lambda h, c: (0, h)),
            scratch_shapes=[
                pltpu.VMEM((2, 2, SKV, DH), jnp.float32),
                pltpu.SemaphoreType.DMA((2, 2)),
            ],
        ),
        out_shape=jax.ShapeDtypeStruct((SQ, H_PER * DH), jnp.bfloat16),
    )(col_base, x, Wq, K_ext, V_ext)


HCHUNK = SQ // 2 // N_DEV


def _allreduce_body(row_ref, ctx_ref, wo_ref, out_ref, send_buf, recv_buf, sems):
    del row_ref
    my = lax.axis_index("i")
    left = lax.rem(my + N_DEV - 1, N_DEV)
    right = lax.rem(my + 1, N_DEV)

    wo_bf = wo_ref[...].astype(jnp.bfloat16)

    def rows_of(ring, c):
        return pl.ds(ring * (SQ // 2) + c * HCHUNK, HCHUNK)

    def proj(ring, c):
        return jnp.dot(
            ctx_ref[rows_of(ring, c), :], wo_bf,
            preferred_element_type=jnp.float32,
        )

    def rdma(k, s, src, dev):
        return pltpu.make_async_remote_copy(
            src_ref=src,
            dst_ref=recv_buf.at[k, s],
            send_sem=sems.at[0, k, s],
            recv_sem=sems.at[1, k, s],
            device_id=(dev,),
            device_id_type=pl.DeviceIdType.MESH,
        )

    send_buf[0, 0] = proj(0, my).astype(jnp.bfloat16)
    send_buf[2, 0] = proj(1, my).astype(jnp.bfloat16)

    barrier_sem = pltpu.get_barrier_semaphore()
    for nbr in (left, right):
        pl.semaphore_signal(
            barrier_sem, inc=1,
            device_id=(nbr,), device_id_type=pl.DeviceIdType.MESH,
        )
    pl.semaphore_wait(barrier_sem, 2)

    for s in range(N_DEV - 1):
        rc_a = lax.rem(my - s - 1 + N_DEV, N_DEV)
        rc_b = lax.rem(my + s + 1, N_DEV)
        ra = rdma(0, s, send_buf.at[0, s], right)
        rb = rdma(2, s, send_buf.at[2, s], left)
        ra.start()
        rb.start()
        pv_a = proj(0, rc_a)
        pv_b = proj(1, rc_b)
        ra.wait()
        rb.wait()
        sum_a = pv_a + recv_buf[0, s].astype(jnp.float32)
        sum_b = pv_b + recv_buf[2, s].astype(jnp.float32)
        if s < N_DEV - 2:
            send_buf[0, s + 1] = sum_a.astype(jnp.bfloat16)
            send_buf[2, s + 1] = sum_b.astype(jnp.bfloat16)
        else:
            out_ref[0, rows_of(0, rc_a), :] = sum_a
            out_ref[0, rows_of(1, rc_b), :] = sum_b
            send_buf[1, 0] = sum_a.astype(jnp.bfloat16)
            send_buf[3, 0] = sum_b.astype(jnp.bfloat16)

    for t in range(N_DEV - 1):
        src_a = send_buf.at[1, 0] if t == 0 else recv_buf.at[1, t - 1]
        src_b = send_buf.at[3, 0] if t == 0 else recv_buf.at[3, t - 1]
        ra = rdma(1, t, src_a, right)
        rb = rdma(3, t, src_b, left)
        ra.start()
        rb.start()
        if t > 0:
            out_ref[0, rows_of(0, lax.rem(my - t + 1 + N_DEV, N_DEV)), :] = (
                recv_buf[1, t - 1].astype(jnp.float32)
            )
            out_ref[0, rows_of(1, lax.rem(my + t - 1, N_DEV)), :] = (
                recv_buf[3, t - 1].astype(jnp.float32)
            )
        ra.wait()
        rb.wait()
    last = N_DEV - 2
    out_ref[0, rows_of(0, lax.rem(my - last + N_DEV, N_DEV)), :] = (
        recv_buf[1, last].astype(jnp.float32)
    )
    out_ref[0, rows_of(1, lax.rem(my + last, N_DEV)), :] = (
        recv_buf[3, last].astype(jnp.float32)
    )


def _project_allreduce(ctx, Wo, row_base):
    return pl.pallas_call(
        _allreduce_body,
        grid_spec=pltpu.PrefetchScalarGridSpec(
            num_scalar_prefetch=1,
            grid=(1,),
            in_specs=[
                pl.BlockSpec((SQ, H_PER * DH), lambda i, r: (0, 0)),
                pl.BlockSpec((H_PER * DH, D_MODEL), lambda i, r: (r[0], 0)),
            ],
            out_specs=pl.BlockSpec((1, SQ, D_MODEL), lambda i, r: (0, 0, 0)),
            scratch_shapes=[
                pltpu.VMEM((4, N_DEV - 1, HCHUNK, D_MODEL), jnp.bfloat16),
                pltpu.VMEM((4, N_DEV - 1, HCHUNK, D_MODEL), jnp.bfloat16),
                pltpu.SemaphoreType.DMA((2, 4, N_DEV - 1)),
            ],
        ),
        out_shape=jax.ShapeDtypeStruct((1, SQ, D_MODEL), jnp.float32),
        compiler_params=pltpu.CompilerParams(collective_id=0),
    )(row_base, ctx, Wo)


def kernel(x, Wq, K_ext, V_ext, Wo):
    idx = lax.axis_index("i")
    col_base = jnp.reshape(idx * H_PER, (1,)).astype(jnp.int32)
    row_base = jnp.reshape(idx, (1,)).astype(jnp.int32)
    ctx = _attention(x, Wq, K_ext, V_ext, col_base)
    return _project_allreduce(ctx, Wo, row_base)


# device time: 155100 ns/iter; 1.0468x vs baseline; 1.0468x over previous
import functools

import jax
import jax.numpy as jnp
from jax import lax
from jax.experimental import pallas as pl
from jax.experimental.pallas import tpu as pltpu

N_DEV = 8
SQ = 2048
SKV = 2048
D_MODEL = 1024
H_PER = 8
DH = 128
SCALE = 0.08838834764831843
CHUNK = SQ // N_DEV


QT = 256
N_QT = SQ // QT
KT = 256
N_KT = SKV // KT


def _key_tiles(qt: int) -> list[int]:
    if qt == 0:
        return list(range(N_KT))
    tiles = {0, qt - 1, qt, qt + 1}
    return sorted(t for t in tiles if 0 <= t < N_KT)


def _attn_body(x_ref, wq_ref, k_hbm, v_hbm, ctx_ref, kv_buf, kv_sems):
    h = pl.program_id(0)
    slot = lax.rem(h, 2)

    def kv_copy(head, sl):
        return (
            pltpu.make_async_copy(
                k_hbm.at[0, :, head, :], kv_buf.at[sl, 0], kv_sems.at[sl, 0]
            ),
            pltpu.make_async_copy(
                v_hbm.at[0, :, head, :], kv_buf.at[sl, 1], kv_sems.at[sl, 1]
            ),
        )

    @pl.when(h == 0)
    def _():
        for c in kv_copy(h, slot):
            c.start()

    @pl.when(h + 1 < H_PER)
    def _():
        for c in kv_copy(h + 1, lax.rem(h + 1, 2)):
            c.start()

    xm = x_ref[0]
    q = jnp.dot(xm, wq_ref[...], preferred_element_type=jnp.float32)
    for c in kv_copy(h, slot):
        c.wait()
    k = kv_buf[slot, 0]
    v = kv_buf[slot, 1]
    for qt in range(N_QT):
        tiles = _key_tiles(qt)
        q_t = q[qt * QT:(qt + 1) * QT, :]
        k_sel = jnp.concatenate([k[t * KT:(t + 1) * KT, :] for t in tiles], 0)
        v_sel = jnp.concatenate([v[t * KT:(t + 1) * KT, :] for t in tiles], 0)
        s = lax.dot_general(
            q_t, k_sel, (((1,), (1,)), ((), ())),
            preferred_element_type=jnp.float32,
        ) * SCALE
        qi = qt * QT + lax.broadcasted_iota(jnp.int32, s.shape, 0)
        ki = jnp.concatenate(
            [t * KT + lax.broadcasted_iota(jnp.int32, (QT, KT), 1) for t in tiles],
            axis=1,
        )
        mask = (jnp.abs(qi - ki) <= 128) | (ki < 32) | (qi < 32)
        s = jnp.where(mask, s, -1e9)
        m = jnp.max(s, axis=1, keepdims=True)
        w = jnp.exp(s - m)
        w = w / jnp.sum(w, axis=1, keepdims=True)
        ctx_ref[qt * QT:(qt + 1) * QT, :] = jnp.dot(
            w, v_sel, preferred_element_type=jnp.float32
        ).astype(jnp.bfloat16)


def _attention(x, Wq_l, K_ext, V_ext):
    return pl.pallas_call(
        _attn_body,
        grid=(H_PER,),
        in_specs=[
            pl.BlockSpec((1, SQ, D_MODEL), lambda h: (0, 0, 0)),
            pl.BlockSpec((D_MODEL, DH), lambda h: (0, h)),
            pl.BlockSpec(memory_space=pltpu.MemorySpace.HBM),
            pl.BlockSpec(memory_space=pltpu.MemorySpace.HBM),
        ],
        out_specs=pl.BlockSpec((SQ, DH), lambda h: (0, h)),
        out_shape=jax.ShapeDtypeStruct((SQ, H_PER * DH), jnp.bfloat16),
        scratch_shapes=[
            pltpu.VMEM((2, 2, SKV, DH), jnp.float32),
            pltpu.SemaphoreType.DMA((2, 2)),
        ],
    )(x, Wq_l, K_ext, V_ext)


HCHUNK = SQ // 2 // N_DEV


def _allreduce_body(ctx_ref, wo_ref, out_ref, send_buf, recv_buf, sems):
    my = lax.axis_index("i")
    left = lax.rem(my + N_DEV - 1, N_DEV)
    right = lax.rem(my + 1, N_DEV)

    wo_bf = wo_ref[...].astype(jnp.bfloat16)

    def rows_of(ring, c):
        return pl.ds(ring * (SQ // 2) + c * HCHUNK, HCHUNK)

    def proj(ring, c):
        return jnp.dot(
            ctx_ref[rows_of(ring, c), :], wo_bf,
            preferred_element_type=jnp.float32,
        )

    def rdma(k, s, src, dev):
        return pltpu.make_async_remote_copy(
            src_ref=src,
            dst_ref=recv_buf.at[k, s],
            send_sem=sems.at[0, k, s],
            recv_sem=sems.at[1, k, s],
            device_id=(dev,),
            device_id_type=pl.DeviceIdType.MESH,
        )

    send_buf[0, 0] = proj(0, my).astype(jnp.bfloat16)
    send_buf[2, 0] = proj(1, my).astype(jnp.bfloat16)

    barrier_sem = pltpu.get_barrier_semaphore()
    for nbr in (left, right):
        pl.semaphore_signal(
            barrier_sem, inc=1,
            device_id=(nbr,), device_id_type=pl.DeviceIdType.MESH,
        )
    pl.semaphore_wait(barrier_sem, 2)

    for s in range(N_DEV - 1):
        rc_a = lax.rem(my - s - 1 + N_DEV, N_DEV)
        rc_b = lax.rem(my + s + 1, N_DEV)
        ra = rdma(0, s, send_buf.at[0, s], right)
        rb = rdma(2, s, send_buf.at[2, s], left)
        ra.start()
        rb.start()
        pv_a = proj(0, rc_a)
        pv_b = proj(1, rc_b)
        ra.wait()
        rb.wait()
        sum_a = pv_a + recv_buf[0, s].astype(jnp.float32)
        sum_b = pv_b + recv_buf[2, s].astype(jnp.float32)
        if s < N_DEV - 2:
            send_buf[0, s + 1] = sum_a.astype(jnp.bfloat16)
            send_buf[2, s + 1] = sum_b.astype(jnp.bfloat16)
        else:
            out_ref[0, rows_of(0, rc_a), :] = sum_a
            out_ref[0, rows_of(1, rc_b), :] = sum_b
            send_buf[1, 0] = sum_a.astype(jnp.bfloat16)
            send_buf[3, 0] = sum_b.astype(jnp.bfloat16)

    for t in range(N_DEV - 1):
        src_a = send_buf.at[1, 0] if t == 0 else recv_buf.at[1, t - 1]
        src_b = send_buf.at[3, 0] if t == 0 else recv_buf.at[3, t - 1]
        ra = rdma(1, t, src_a, right)
        rb = rdma(3, t, src_b, left)
        ra.start()
        rb.start()
        if t > 0:
            out_ref[0, rows_of(0, lax.rem(my - t + 1 + N_DEV, N_DEV)), :] = (
                recv_buf[1, t - 1].astype(jnp.float32)
            )
            out_ref[0, rows_of(1, lax.rem(my + t - 1, N_DEV)), :] = (
                recv_buf[3, t - 1].astype(jnp.float32)
            )
        ra.wait()
        rb.wait()
    last = N_DEV - 2
    out_ref[0, rows_of(0, lax.rem(my - last + N_DEV, N_DEV)), :] = (
        recv_buf[1, last].astype(jnp.float32)
    )
    out_ref[0, rows_of(1, lax.rem(my + last, N_DEV)), :] = (
        recv_buf[3, last].astype(jnp.float32)
    )


def _project_allreduce(ctx, Wo_l):
    return pl.pallas_call(
        _allreduce_body,
        out_shape=jax.ShapeDtypeStruct((1, SQ, D_MODEL), jnp.float32),
        in_specs=[
            pl.BlockSpec(memory_space=pltpu.VMEM),
            pl.BlockSpec(memory_space=pltpu.VMEM),
        ],
        out_specs=pl.BlockSpec(memory_space=pltpu.VMEM),
        scratch_shapes=[
            pltpu.VMEM((4, N_DEV - 1, HCHUNK, D_MODEL), jnp.bfloat16),
            pltpu.VMEM((4, N_DEV - 1, HCHUNK, D_MODEL), jnp.bfloat16),
            pltpu.SemaphoreType.DMA((2, 4, N_DEV - 1)),
        ],
        compiler_params=pltpu.CompilerParams(collective_id=0),
    )(ctx, Wo_l)


def kernel(x, Wq, K_ext, V_ext, Wo):
    idx = lax.axis_index("i")
    Wq_l = lax.dynamic_slice(Wq, (0, idx * H_PER * DH), (D_MODEL, H_PER * DH))
    Wo_l = lax.dynamic_slice(Wo, (idx * H_PER * DH, 0), (H_PER * DH, D_MODEL))
    ctx = _attention(x, Wq_l, K_ext, V_ext)
    return _project_allreduce(ctx, Wo_l)
